# baseline (speedup 1.0000x reference)
import functools

import jax
import jax.numpy as jnp
from jax.experimental import pallas as pl
from jax.experimental.pallas import tpu as pltpu

_LAM = 1e-4
_ROWS = 8


def _simam_body(x_ref, o_ref, *, inv_dhw, inv_n, lam):
    x = x_ref[...]
    s1 = jnp.sum(x, axis=1, keepdims=True)
    s2 = jnp.sum(x * x, axis=1, keepdims=True)
    mu = s1 * inv_dhw
    d2sum = jnp.maximum(s2 - s1 * mu, 0.0)
    inv2 = 0.5 * pl.reciprocal(4.0 * (d2sum * inv_n + lam), approx=False)
    xm = x - mu
    t = jnp.tanh(xm * xm * inv2 + 0.25)
    hx = 0.5 * x
    o_ref[...] = hx + hx * t


def kernel(x):
    B, C, D, H, W = x.shape
    DHW = D * H * W
    x2 = x.reshape(B * C, DHW)
    rows = _ROWS
    body = functools.partial(
        _simam_body,
        inv_dhw=1.0 / float(DHW),
        inv_n=1.0 / float(DHW - 1),
        lam=_LAM,
    )
    out = pl.pallas_call(
        body,
        out_shape=jax.ShapeDtypeStruct((B * C, DHW), x.dtype),
        grid=(B * C // rows,),
        in_specs=[pl.BlockSpec((rows, DHW), lambda i: (i, 0))],
        out_specs=pl.BlockSpec((rows, DHW), lambda i: (i, 0)),
        compiler_params=pltpu.CompilerParams(
            dimension_semantics=("parallel",),
            vmem_limit_bytes=64 * 1024 * 1024,
        ),
    )(x2)
    return out.reshape(B, C, D, H, W)


# speedup vs baseline: 2.7001x; 2.7001x over previous
import functools

import jax
import jax.numpy as jnp
from jax.experimental import pallas as pl
from jax.experimental.pallas import tpu as pltpu

_LAM = 1e-4
_CT = 32


def _simam_body(x_ref, o_ref, *, inv_dhw, inv_n, lam):
    x = x_ref[0]
    s1 = jnp.sum(x, axis=1, keepdims=True)
    s2 = jnp.sum(x * x, axis=1, keepdims=True)
    mu = s1 * inv_dhw
    d2sum = jnp.maximum(s2 - s1 * mu, 0.0)
    inv2 = 0.5 * pl.reciprocal(4.0 * (d2sum * inv_n + lam), approx=False)
    xm = x - mu
    t = jnp.tanh(xm * xm * inv2 + 0.25)
    hx = 0.5 * x
    o_ref[...] = (hx + hx * t)[None, :, :]


def kernel(x):
    B, C, D, H, W = x.shape
    DHW = D * H * W
    x3 = x.reshape(B, C, DHW)
    ct = _CT
    body = functools.partial(
        _simam_body,
        inv_dhw=1.0 / float(DHW),
        inv_n=1.0 / float(DHW - 1),
        lam=_LAM,
    )
    out = pl.pallas_call(
        body,
        out_shape=jax.ShapeDtypeStruct((B, C, DHW), x.dtype),
        grid=(B, C // ct),
        in_specs=[pl.BlockSpec((1, ct, DHW), lambda b, c: (b, c, 0))],
        out_specs=pl.BlockSpec((1, ct, DHW), lambda b, c: (b, c, 0)),
        compiler_params=pltpu.CompilerParams(
            dimension_semantics=("parallel", "arbitrary"),
            vmem_limit_bytes=64 * 1024 * 1024,
        ),
    )(x3)
    return out.reshape(B, C, D, H, W)


# speedup vs baseline: 2.9594x; 1.0961x over previous
import functools

import jax
import jax.numpy as jnp
from jax.experimental import pallas as pl
from jax.experimental.pallas import tpu as pltpu

_LAM = 1e-4
_CT = 64


def _simam_body(x_ref, o_ref, *, inv_dhw, inv_n, lam):
    x = x_ref[0]
    s1 = jnp.sum(x, axis=1, keepdims=True)
    s2 = jnp.sum(x * x, axis=1, keepdims=True)
    mu = s1 * inv_dhw
    d2sum = jnp.maximum(s2 - s1 * mu, 0.0)
    inv2 = 0.5 * pl.reciprocal(4.0 * (d2sum * inv_n + lam), approx=False)
    xm = x - mu
    t = jnp.tanh(xm * xm * inv2 + 0.25)
    hx = 0.5 * x
    o_ref[...] = (hx + hx * t)[None, :, :]


def kernel(x):
    B, C, D, H, W = x.shape
    DHW = D * H * W
    x3 = x.reshape(B, C, DHW)
    ct = _CT
    body = functools.partial(
        _simam_body,
        inv_dhw=1.0 / float(DHW),
        inv_n=1.0 / float(DHW - 1),
        lam=_LAM,
    )
    out = pl.pallas_call(
        body,
        out_shape=jax.ShapeDtypeStruct((B, C, DHW), x.dtype),
        grid=(B, C // ct),
        in_specs=[pl.BlockSpec((1, ct, DHW), lambda b, c: (b, c, 0))],
        out_specs=pl.BlockSpec((1, ct, DHW), lambda b, c: (b, c, 0)),
        compiler_params=pltpu.CompilerParams(
            dimension_semantics=("parallel", "arbitrary"),
            vmem_limit_bytes=64 * 1024 * 1024,
        ),
    )(x3)
    return out.reshape(B, C, D, H, W)


# speedup vs baseline: 3.1988x; 1.0809x over previous
import functools

import jax
import jax.numpy as jnp
from jax.experimental import pallas as pl
from jax.experimental.pallas import tpu as pltpu

_LAM = 1e-4
_BT = 2


def _simam_body(x_ref, o_ref, *, inv_dhw, inv_n, lam):
    x = x_ref[...]
    s1 = jnp.sum(x, axis=2, keepdims=True)
    s2 = jnp.sum(x * x, axis=2, keepdims=True)
    mu = s1 * inv_dhw
    d2sum = jnp.maximum(s2 - s1 * mu, 0.0)
    inv2 = 0.5 * pl.reciprocal(4.0 * (d2sum * inv_n + lam), approx=False)
    xm = x - mu
    t = jnp.tanh(xm * xm * inv2 + 0.25)
    hx = 0.5 * x
    o_ref[...] = hx + hx * t


def kernel(x):
    B, C, D, H, W = x.shape
    DHW = D * H * W
    x3 = x.reshape(B, C, DHW)
    bt = _BT
    body = functools.partial(
        _simam_body,
        inv_dhw=1.0 / float(DHW),
        inv_n=1.0 / float(DHW - 1),
        lam=_LAM,
    )
    out = pl.pallas_call(
        body,
        out_shape=jax.ShapeDtypeStruct((B, C, DHW), x.dtype),
        grid=(B // bt,),
        in_specs=[pl.BlockSpec((bt, C, DHW), lambda b: (b, 0, 0))],
        out_specs=pl.BlockSpec((bt, C, DHW), lambda b: (b, 0, 0)),
        compiler_params=pltpu.CompilerParams(
            dimension_semantics=("parallel",),
            vmem_limit_bytes=64 * 1024 * 1024,
        ),
    )(x3)
    return out.reshape(B, C, D, H, W)


# speedup vs baseline: 3.2769x; 1.0244x over previous
import functools

import jax
import jax.numpy as jnp
from jax.experimental import pallas as pl
from jax.experimental.pallas import tpu as pltpu

_LAM = 1e-4
_BT = 4


def _simam_body(x_ref, o_ref, *, inv_dhw, inv_n, lam):
    x = x_ref[...]
    s1 = jnp.sum(x, axis=2, keepdims=True)
    s2 = jnp.sum(x * x, axis=2, keepdims=True)
    mu = s1 * inv_dhw
    d2sum = jnp.maximum(s2 - s1 * mu, 0.0)
    inv2 = 0.5 * pl.reciprocal(4.0 * (d2sum * inv_n + lam), approx=False)
    xm = x - mu
    t = jnp.tanh(xm * xm * inv2 + 0.25)
    hx = 0.5 * x
    o_ref[...] = hx + hx * t


def kernel(x):
    B, C, D, H, W = x.shape
    DHW = D * H * W
    x3 = x.reshape(B, C, DHW)
    bt = _BT
    body = functools.partial(
        _simam_body,
        inv_dhw=1.0 / float(DHW),
        inv_n=1.0 / float(DHW - 1),
        lam=_LAM,
    )
    out = pl.pallas_call(
        body,
        out_shape=jax.ShapeDtypeStruct((B, C, DHW), x.dtype),
        grid=(B // bt,),
        in_specs=[pl.BlockSpec((bt, C, DHW), lambda b: (b, 0, 0))],
        out_specs=pl.BlockSpec((bt, C, DHW), lambda b: (b, 0, 0)),
        compiler_params=pltpu.CompilerParams(
            dimension_semantics=("parallel",),
            vmem_limit_bytes=64 * 1024 * 1024,
        ),
    )(x3)
    return out.reshape(B, C, D, H, W)


# speedup vs baseline: 6.6360x; 2.0251x over previous
import functools

import jax
import jax.numpy as jnp
from jax.experimental import pallas as pl
from jax.experimental.pallas import tpu as pltpu

_LAM = 1e-4
_BT = 1


def _simam_body(x_ref, o_ref, *, inv_dhw, inv_n, lam):
    x = x_ref[...]
    s1 = jnp.sum(x, axis=1, keepdims=True)
    s2 = jnp.sum(x * x, axis=1, keepdims=True)
    mu = s1 * inv_dhw
    d2sum = jnp.maximum(s2 - s1 * mu, 0.0)
    inv2 = 0.5 * pl.reciprocal(4.0 * (d2sum * inv_n + lam), approx=False)
    xm = x - mu
    t = jnp.tanh(xm * xm * inv2 + 0.25)
    hx = 0.5 * x
    o_ref[...] = hx + hx * t


def kernel(x):
    B, C, D, H, W = x.shape
    DHW = D * H * W
    xt = jnp.transpose(x, (0, 2, 3, 4, 1)).reshape(B, DHW, C)
    bt = _BT
    body = functools.partial(
        _simam_body,
        inv_dhw=1.0 / float(DHW),
        inv_n=1.0 / float(DHW - 1),
        lam=_LAM,
    )
    out = pl.pallas_call(
        body,
        out_shape=jax.ShapeDtypeStruct((B, DHW, C), x.dtype),
        grid=(B // bt,),
        in_specs=[pl.BlockSpec((bt, DHW, C), lambda b: (b, 0, 0))],
        out_specs=pl.BlockSpec((bt, DHW, C), lambda b: (b, 0, 0)),
        compiler_params=pltpu.CompilerParams(
            dimension_semantics=("parallel",),
            vmem_limit_bytes=64 * 1024 * 1024,
        ),
    )(xt)
    return jnp.transpose(out.reshape(B, D, H, W, C), (0, 4, 1, 2, 3))


# speedup vs baseline: 7.6978x; 1.1600x over previous
import functools

import jax
import jax.numpy as jnp
from jax.experimental import pallas as pl
from jax.experimental.pallas import tpu as pltpu

_LAM = 1e-4
_BT = 2


def _simam_body(x_ref, o_ref, *, inv_dhw, inv_n, lam):
    x = x_ref[...]
    s1 = jnp.sum(x, axis=1, keepdims=True)
    s2 = jnp.sum(x * x, axis=1, keepdims=True)
    mu = s1 * inv_dhw
    d2sum = jnp.maximum(s2 - s1 * mu, 0.0)
    inv2 = 0.5 * pl.reciprocal(4.0 * (d2sum * inv_n + lam), approx=False)
    xm = x - mu
    t = jnp.tanh(xm * xm * inv2 + 0.25)
    hx = 0.5 * x
    o_ref[...] = hx + hx * t


def kernel(x):
    B, C, D, H, W = x.shape
    DHW = D * H * W
    xt = jnp.transpose(x, (0, 2, 3, 4, 1)).reshape(B, DHW, C)
    bt = _BT
    body = functools.partial(
        _simam_body,
        inv_dhw=1.0 / float(DHW),
        inv_n=1.0 / float(DHW - 1),
        lam=_LAM,
    )
    out = pl.pallas_call(
        body,
        out_shape=jax.ShapeDtypeStruct((B, DHW, C), x.dtype),
        grid=(B // bt,),
        in_specs=[pl.BlockSpec((bt, DHW, C), lambda b: (b, 0, 0))],
        out_specs=pl.BlockSpec((bt, DHW, C), lambda b: (b, 0, 0)),
        compiler_params=pltpu.CompilerParams(
            dimension_semantics=("parallel",),
            vmem_limit_bytes=64 * 1024 * 1024,
        ),
    )(xt)
    return jnp.transpose(out.reshape(B, D, H, W, C), (0, 4, 1, 2, 3))


# speedup vs baseline: 8.0864x; 1.0505x over previous
import functools

import jax
import jax.numpy as jnp
from jax.experimental import pallas as pl
from jax.experimental.pallas import tpu as pltpu

_LAM = 1e-4
_BT = 4


def _simam_body(x_ref, o_ref, *, inv_dhw, inv_n, lam):
    x = x_ref[...]
    s1 = jnp.sum(x, axis=1, keepdims=True)
    s2 = jnp.sum(x * x, axis=1, keepdims=True)
    mu = s1 * inv_dhw
    d2sum = jnp.maximum(s2 - s1 * mu, 0.0)
    inv2 = 0.5 * pl.reciprocal(4.0 * (d2sum * inv_n + lam), approx=False)
    xm = x - mu
    t = jnp.tanh(xm * xm * inv2 + 0.25)
    hx = 0.5 * x
    o_ref[...] = hx + hx * t


def kernel(x):
    B, C, D, H, W = x.shape
    DHW = D * H * W
    xt = jnp.transpose(x, (0, 2, 3, 4, 1)).reshape(B, DHW, C)
    bt = _BT
    body = functools.partial(
        _simam_body,
        inv_dhw=1.0 / float(DHW),
        inv_n=1.0 / float(DHW - 1),
        lam=_LAM,
    )
    out = pl.pallas_call(
        body,
        out_shape=jax.ShapeDtypeStruct((B, DHW, C), x.dtype),
        grid=(B // bt,),
        in_specs=[pl.BlockSpec((bt, DHW, C), lambda b: (b, 0, 0))],
        out_specs=pl.BlockSpec((bt, DHW, C), lambda b: (b, 0, 0)),
        compiler_params=pltpu.CompilerParams(
            dimension_semantics=("parallel",),
            vmem_limit_bytes=64 * 1024 * 1024,
        ),
    )(xt)
    return jnp.transpose(out.reshape(B, D, H, W, C), (0, 4, 1, 2, 3))
